# baseline (speedup 1.0000x reference)
import functools

import jax
import jax.numpy as jnp
from jax import lax
from jax.experimental import pallas as pl
from jax.experimental.pallas import tpu as pltpu
from jax.experimental.pallas import tpu_sc as plsc

B = 16384
D = 128
NC = 2
NS = 16
NW = NC * NS
BPW = B // NW
CH = 128
NCH = BPW // CH


def _make_sc_kernel():
    mesh = plsc.VectorSubcoreMesh(core_axis_name="c", subcore_axis_name="s")

    @functools.partial(
        pl.kernel,
        mesh=mesh,
        out_type=jax.ShapeDtypeStruct((NW, BPW, D), jnp.float32),
        scratch_types=[
            pltpu.VMEM((NCH, CH), jnp.int32),
            pltpu.VMEM((BPW, D), jnp.float32),
            pltpu.SemaphoreType.DMA,
        ],
    )
    def k(idx_hbm, table_hbm, out_hbm, idx_v, rows_v, sem):
        wid = lax.axis_index("s") * NC + lax.axis_index("c")
        pltpu.sync_copy(idx_hbm.at[wid], idx_v)
        copies = [
            pltpu.async_copy(
                table_hbm.at[idx_v.at[j]],
                rows_v.at[pl.ds(j * CH, CH)],
                sem,
            )
            for j in range(NCH)
        ]
        for c in copies:
            c.wait()
        pltpu.sync_copy(rows_v, out_hbm.at[wid])

    return k


_sc_kernel = _make_sc_kernel()


def kernel(state, state_embed):
    idx = state.reshape(NW, NCH, CH)
    out = _sc_kernel(idx, state_embed)
    return out.reshape(B, D)


# speedup vs baseline: 5.6932x; 5.6932x over previous
import functools

import jax
import jax.numpy as jnp
from jax import lax
from jax.experimental import pallas as pl
from jax.experimental.pallas import tpu as pltpu
from jax.experimental.pallas import tpu_sc as plsc

B = 16384
D = 128
NC = 2
NS = 16
NW = NC * NS
BPW = B // NW
CHUNK = 128
NCHUNK = BPW // CHUNK
BLK = 16


def _make_sc_kernel():
    mesh = plsc.VectorSubcoreMesh(core_axis_name="c", subcore_axis_name="s")

    @functools.partial(
        pl.kernel,
        mesh=mesh,
        out_type=jax.ShapeDtypeStruct((NW, BPW, D), jnp.float32),
        scratch_types=[
            pltpu.VMEM((BPW,), jnp.int32),
            pltpu.VMEM((3, D), jnp.float32),
            pltpu.VMEM((BPW, D), jnp.float32),
            pltpu.SemaphoreType.DMA,
        ],
    )
    def k(idx_hbm, table_hbm, out_hbm, idx_v, table_v, rows_v, sem):
        wid = lax.axis_index("s") * NC + lax.axis_index("c")
        pltpu.sync_copy(table_hbm, table_v)
        pltpu.sync_copy(idx_hbm.at[wid], idx_v)

        copies = []
        for j in range(NCHUNK):
            def blk_body(ib):
                base = j * CHUNK + ib * BLK
                v = idx_v[pl.ds(base, BLK)]
                for l in range(BLK):
                    r = v[l]
                    for c in range(D // 16):
                        rows_v[base + l, pl.ds(c * 16, 16)] = (
                            table_v[r, pl.ds(c * 16, 16)]
                        )
            pl.loop(0, CHUNK // BLK)(blk_body)
            copies.append(
                pltpu.async_copy(
                    rows_v.at[pl.ds(j * CHUNK, CHUNK)],
                    out_hbm.at[wid, pl.ds(j * CHUNK, CHUNK)],
                    sem,
                )
            )
        for c in copies:
            c.wait()

    return k


_sc_kernel = _make_sc_kernel()


def kernel(state, state_embed):
    idx = state.reshape(NW, BPW)
    out = _sc_kernel(idx, state_embed)
    return out.reshape(B, D)


# speedup vs baseline: 6.0464x; 1.0620x over previous
import functools

import jax
import jax.numpy as jnp
from jax import lax
from jax.experimental import pallas as pl
from jax.experimental.pallas import tpu as pltpu
from jax.experimental.pallas import tpu_sc as plsc

B = 16384
D = 128
NC = 2
NS = 16
NW = NC * NS
BPW = B // NW
CHUNK = 128
NCHUNK = BPW // CHUNK
BLK = 16


def _bcast_lane(v, l):
    return lax.gather(
        v,
        jnp.full((16, 1), l, jnp.int32),
        lax.GatherDimensionNumbers(
            offset_dims=(), collapsed_slice_dims=(0,), start_index_map=(0,)
        ),
        slice_sizes=(1,),
        mode=lax.GatherScatterMode.PROMISE_IN_BOUNDS,
    )


def _make_sc_kernel():
    mesh = plsc.VectorSubcoreMesh(core_axis_name="c", subcore_axis_name="s")

    @functools.partial(
        pl.kernel,
        mesh=mesh,
        compiler_params=pltpu.CompilerParams(needs_layout_passes=False),
        out_type=jax.ShapeDtypeStruct((NW, BPW, D), jnp.float32),
        scratch_types=[
            pltpu.VMEM((BPW,), jnp.int32),
            pltpu.VMEM((3 * D,), jnp.float32),
            pltpu.VMEM((BPW, D), jnp.float32),
            pltpu.SemaphoreType.DMA,
        ],
    )
    def k(idx_hbm, table_hbm, out_hbm, idx_v, table_v, rows_v, sem):
        wid = lax.axis_index("s") * NC + lax.axis_index("c")
        pltpu.sync_copy(table_hbm, table_v)
        pltpu.sync_copy(idx_hbm.at[wid], idx_v)

        lane = lax.iota(jnp.int32, 16)
        colconst = [c * 16 + lane for c in range(D // 16)]

        copies = []
        for j in range(NCHUNK):
            def blk_body(ib):
                base = j * CHUNK + ib * BLK
                rowbase = idx_v[pl.ds(base, BLK)] * D
                for l in range(BLK):
                    bl = _bcast_lane(rowbase, l)
                    for c in range(D // 16):
                        val = plsc.load_gather(table_v, [bl + colconst[c]])
                        rows_v[base + l, pl.ds(c * 16, 16)] = val
            pl.loop(0, CHUNK // BLK)(blk_body)
            copies.append(
                pltpu.async_copy(
                    rows_v.at[pl.ds(j * CHUNK, CHUNK)],
                    out_hbm.at[wid, pl.ds(j * CHUNK, CHUNK)],
                    sem,
                )
            )
        for c in copies:
            c.wait()

    return k


_sc_kernel = _make_sc_kernel()


def kernel(state, state_embed):
    idx = state.reshape(NW, BPW)
    out = _sc_kernel(idx, state_embed.reshape(3 * D))
    return out.reshape(B, D)


# speedup vs baseline: 8.8685x; 1.4667x over previous
import functools

import jax
import jax.numpy as jnp
from jax import lax
from jax.experimental import pallas as pl
from jax.experimental.pallas import tpu as pltpu
from jax.experimental.pallas import tpu_sc as plsc

B = 16384
D = 128
NC = 2
NS = 16
NW = NC * NS
BPW = B // NW
CHUNK = 128
NCHUNK = BPW // CHUNK
BLK = 16
CG = D // 16


def _bcast_lane(v, l):
    return lax.gather(
        v,
        jnp.full((16, 1), l, jnp.int32),
        lax.GatherDimensionNumbers(
            offset_dims=(), collapsed_slice_dims=(0,), start_index_map=(0,)
        ),
        slice_sizes=(1,),
        mode=lax.GatherScatterMode.PROMISE_IN_BOUNDS,
    )


def _make_sc_kernel():
    mesh = plsc.VectorSubcoreMesh(core_axis_name="c", subcore_axis_name="s")

    @functools.partial(
        pl.kernel,
        mesh=mesh,
        compiler_params=pltpu.CompilerParams(needs_layout_passes=False),
        out_type=jax.ShapeDtypeStruct((NW, BPW, D), jnp.float32),
        scratch_types=[
            pltpu.VMEM((BPW,), jnp.int32),
            pltpu.VMEM((3, D), jnp.float32),
            pltpu.VMEM((BPW, D), jnp.float32),
            pltpu.SemaphoreType.DMA,
        ],
    )
    def k(idx_hbm, table_hbm, out_hbm, idx_v, table_v, rows_v, sem):
        wid = lax.axis_index("s") * NC + lax.axis_index("c")
        pltpu.sync_copy(table_hbm, table_v)
        pltpu.sync_copy(idx_hbm.at[wid], idx_v)

        tv = [[table_v[r, pl.ds(c * 16, 16)] for c in range(CG)]
              for r in range(3)]

        copies = []
        for j in range(NCHUNK):
            def blk_body(ib):
                base = j * CHUNK + ib * BLK
                idxv = idx_v[pl.ds(base, BLK)]
                for l in range(BLK):
                    b = _bcast_lane(idxv, l)
                    m0 = b == 0
                    m1 = b == 1
                    for c in range(CG):
                        val = jnp.where(
                            m0, tv[0][c], jnp.where(m1, tv[1][c], tv[2][c])
                        )
                        rows_v[base + l, pl.ds(c * 16, 16)] = val
            pl.loop(0, CHUNK // BLK)(blk_body)
            copies.append(
                pltpu.async_copy(
                    rows_v.at[pl.ds(j * CHUNK, CHUNK)],
                    out_hbm.at[wid, pl.ds(j * CHUNK, CHUNK)],
                    sem,
                )
            )
        for c in copies:
            c.wait()

    return k


_sc_kernel = _make_sc_kernel()


def kernel(state, state_embed):
    idx = state.reshape(NW, BPW)
    out = _sc_kernel(idx, state_embed)
    return out.reshape(B, D)


# speedup vs baseline: 9.0662x; 1.0223x over previous
import functools

import jax
import jax.numpy as jnp
from jax import lax
from jax.experimental import pallas as pl
from jax.experimental.pallas import tpu as pltpu
from jax.experimental.pallas import tpu_sc as plsc

B = 16384
D = 128
NC = 2
NS = 16
NW = NC * NS
BPW = B // NW
CHUNK = 128
NCHUNK = BPW // CHUNK
BLK = 16
CG = D // 16


def _bcast_lane(v, l):
    return lax.gather(
        v,
        jnp.full((16, 1), l, jnp.int32),
        lax.GatherDimensionNumbers(
            offset_dims=(), collapsed_slice_dims=(0,), start_index_map=(0,)
        ),
        slice_sizes=(1,),
        mode=lax.GatherScatterMode.PROMISE_IN_BOUNDS,
    )


def _make_sc_kernel():
    mesh = plsc.VectorSubcoreMesh(core_axis_name="c", subcore_axis_name="s")

    @functools.partial(
        pl.kernel,
        mesh=mesh,
        compiler_params=pltpu.CompilerParams(needs_layout_passes=False),
        out_type=jax.ShapeDtypeStruct((NW, BPW, D), jnp.float32),
        scratch_types=[
            pltpu.VMEM((BPW,), jnp.int32),
            pltpu.VMEM((3, D), jnp.float32),
            pltpu.VMEM((BPW, D), jnp.float32),
            pltpu.SemaphoreType.DMA,
            pltpu.SemaphoreType.DMA,
        ],
    )
    def k(idx_hbm, table_hbm, out_hbm, idx_v, table_v, rows_v, sem, sem_in):
        wid = lax.axis_index("s") * NC + lax.axis_index("c")
        c_tab = pltpu.async_copy(table_hbm, table_v, sem_in)
        c_idx = pltpu.async_copy(idx_hbm.at[wid], idx_v, sem_in)
        c_tab.wait()
        c_idx.wait()

        tv = [[table_v[r, pl.ds(c * 16, 16)] for c in range(CG)]
              for r in range(3)]

        copies = []
        for j in range(NCHUNK):
            def blk_body(ib):
                base = j * CHUNK + ib * BLK
                idxv = idx_v[pl.ds(base, BLK)]
                for l in range(BLK):
                    b = _bcast_lane(idxv, l)
                    m0 = b == 0
                    m1 = b == 1
                    for c in range(CG):
                        val = jnp.where(
                            m0, tv[0][c], jnp.where(m1, tv[1][c], tv[2][c])
                        )
                        rows_v[base + l, pl.ds(c * 16, 16)] = val
            pl.loop(0, CHUNK // BLK)(blk_body)
            copies.append(
                pltpu.async_copy(
                    rows_v.at[pl.ds(j * CHUNK, CHUNK)],
                    out_hbm.at[wid, pl.ds(j * CHUNK, CHUNK)],
                    sem,
                )
            )
        for c in copies:
            c.wait()

    return k


_sc_kernel = _make_sc_kernel()


def kernel(state, state_embed):
    idx = state.reshape(NW, BPW)
    out = _sc_kernel(idx, state_embed)
    return out.reshape(B, D)
